# Initial kernel scaffold; baseline (speedup 1.0000x reference)
import jax
import jax.numpy as jnp
from jax.experimental import pallas as pl


def kernel(boxes, box_scores):
    raise NotImplementedError("write your pallas kernel here")


# speedup vs baseline: 2.2685x; 2.2685x over previous
import functools

import jax
import jax.numpy as jnp
from jax import lax
from jax.experimental import pallas as pl
from jax.experimental.pallas import tpu as pltpu

_MAX_BOXES = 20
_SCORE_THRESHOLD = 0.3
_IOU_THRESHOLD = 0.1
_NUM_CLASSES = 80
_N_BOXES = 20000
_CBLK = 16


def _nms_body(scores_ref, boxes_ref, out_ref, s_ref):
    n = _N_BOXES
    neg_inf = jnp.float32(-jnp.inf)

    st = scores_ref[...]
    s_ref[...] = jnp.where(st >= _SCORE_THRESHOLD, st, neg_inf)

    y1 = boxes_ref[0:1, :]
    x1 = boxes_ref[1:2, :]
    y2 = boxes_ref[2:3, :]
    x2 = boxes_ref[3:4, :]
    area_b = (y2 - y1) * (x2 - x1)

    iota_n = lax.broadcasted_iota(jnp.int32, (_CBLK, n), 1)

    def body(t, _):
        s = s_ref[...]
        m = jnp.max(s, axis=1, keepdims=True)
        eq = s == m
        idx = jnp.min(jnp.where(eq, iota_n, n), axis=1, keepdims=True)
        ok = m > neg_inf

        onehot = iota_n == idx
        sel_y1 = jnp.sum(jnp.where(onehot, y1, 0.0), axis=1, keepdims=True)
        sel_x1 = jnp.sum(jnp.where(onehot, x1, 0.0), axis=1, keepdims=True)
        sel_y2 = jnp.sum(jnp.where(onehot, y2, 0.0), axis=1, keepdims=True)
        sel_x2 = jnp.sum(jnp.where(onehot, x2, 0.0), axis=1, keepdims=True)

        iy1 = jnp.maximum(sel_y1, y1)
        ix1 = jnp.maximum(sel_x1, x1)
        iy2 = jnp.minimum(sel_y2, y2)
        ix2 = jnp.minimum(sel_x2, x2)
        inter = jnp.maximum(0.0, iy2 - iy1) * jnp.maximum(0.0, ix2 - ix1)
        area_a = (sel_y2 - sel_y1) * (sel_x2 - sel_x1)
        iou = inter / (area_a + area_b - inter + jnp.float32(1e-9))
        suppress = (iou > _IOU_THRESHOLD) & ok

        s_ref[...] = jnp.where(suppress | onehot, neg_inf, s)

        out_row = jnp.where(ok, idx, -1).astype(jnp.int32)
        out_ref[0, pl.ds(t, 1), :] = out_row.reshape(1, _CBLK)
        return 0

    lax.fori_loop(0, _MAX_BOXES, body, 0, unroll=True)


@jax.jit
def kernel(boxes, box_scores):
    scores_t = box_scores.T
    boxes_t = boxes.T

    n_blocks = _NUM_CLASSES // _CBLK
    nms_idx_blocks = pl.pallas_call(
        _nms_body,
        grid=(n_blocks,),
        in_specs=[
            pl.BlockSpec((_CBLK, _N_BOXES), lambda i: (i, 0)),
            pl.BlockSpec((4, _N_BOXES), lambda i: (0, 0)),
        ],
        out_specs=pl.BlockSpec((1, _MAX_BOXES, _CBLK), lambda i: (i, 0, 0)),
        out_shape=jax.ShapeDtypeStruct((n_blocks, _MAX_BOXES, _CBLK), jnp.int32),
        scratch_shapes=[pltpu.VMEM((_CBLK, _N_BOXES), jnp.float32)],
    )(scores_t, boxes_t)

    nms_idx = nms_idx_blocks.transpose(0, 2, 1).reshape(_NUM_CLASSES, _MAX_BOXES)

    classes = jnp.broadcast_to(
        jnp.arange(_NUM_CLASSES, dtype=jnp.int32)[:, None], nms_idx.shape
    )
    batch = jnp.zeros_like(nms_idx)
    valid = (nms_idx >= 0).reshape(-1, 1)
    nms_final = jnp.stack([batch, classes, nms_idx], axis=-1).reshape(-1, 3)
    nms_final = jnp.where(valid, nms_final, -1)
    return boxes[None], scores_t[None], nms_final[None]
